# Initial kernel scaffold; baseline (speedup 1.0000x reference)
import jax
import jax.numpy as jnp
from jax.experimental import pallas as pl


def kernel(x, h, c, edge_index, W_iouf, U_iou, U_f_w, U_f_b, b_iou):
    raise NotImplementedError("write your pallas kernel here")


# speedup vs baseline: 1.4783x; 1.4783x over previous
import functools

import jax
import jax.numpy as jnp
from jax import lax
from jax.experimental import pallas as pl
from jax.experimental.pallas import tpu as pltpu
from jax.experimental.pallas import tpu_sc as plsc

N = 10000
E = 320000
X = 128
H = 128

NC = 2
NS = 16
NW = NC * NS
EPW = E // NW
B = 80
CHUNKS = EPW // B
NP = 10112
RPT = NP // NS

ROWS = 1000
GRID = N // ROWS


def _tc1_body(x_ref, h_ref, w_ref, uf_ref, ufb_ref, ioux_ref, fnode_ref,
              huf_ref):
  dn = (((1,), (1,)), ((), ()))
  res = lax.dot_general(x_ref[...], w_ref[...], dn,
                        preferred_element_type=jnp.float32)
  ioux_ref[...] = res[:, : 3 * H]
  fnode_ref[...] = res[:, 3 * H:]
  huf_ref[...] = lax.dot_general(h_ref[...], uf_ref[...], dn,
                                 preferred_element_type=jnp.float32) \
      + ufb_ref[...]


def _tc1(x, h, w_iouf, u_f_w, u_f_b):
  return pl.pallas_call(
      _tc1_body,
      grid=(GRID,),
      in_specs=[
          pl.BlockSpec((ROWS, X), lambda i: (i, 0)),
          pl.BlockSpec((ROWS, H), lambda i: (i, 0)),
          pl.BlockSpec((4 * H, X), lambda i: (0, 0)),
          pl.BlockSpec((H, H), lambda i: (0, 0)),
          pl.BlockSpec((1, H), lambda i: (0, 0)),
      ],
      out_specs=[
          pl.BlockSpec((ROWS, 3 * H), lambda i: (i, 0)),
          pl.BlockSpec((ROWS, H), lambda i: (i, 0)),
          pl.BlockSpec((ROWS, H), lambda i: (i, 0)),
      ],
      out_shape=[
          jax.ShapeDtypeStruct((N, 3 * H), jnp.float32),
          jax.ShapeDtypeStruct((N, H), jnp.float32),
          jax.ShapeDtypeStruct((N, H), jnp.float32),
      ],
  )(x, h, w_iouf, u_f_w, u_f_b)


_MESH = plsc.VectorSubcoreMesh(core_axis_name="c", subcore_axis_name="s")


@functools.partial(
    pl.kernel,
    out_type=jax.ShapeDtypeStruct((NC * NP, H), jnp.float32),
    mesh=_MESH,
    scratch_types=[
        pltpu.VMEM((B,), jnp.int32),
        pltpu.VMEM((B,), jnp.int32),
        pltpu.VMEM((B, H), jnp.float32),
        pltpu.VMEM_SHARED((NP, H), jnp.float32),
        pltpu.SemaphoreType.DMA,
    ],
)
def _sc_htild(h_hbm, src_hbm, dst_hbm, zeros_hbm, out_hbm,
              sidx, didx, rows, acc, sem):
  cid = lax.axis_index("c")
  sid = lax.axis_index("s")
  wid = sid * NC + cid
  pltpu.sync_copy(zeros_hbm.at[pl.ds(sid * RPT, RPT)],
                  acc.at[pl.ds(sid * RPT, RPT)])
  plsc.subcore_barrier()
  base = wid * EPW

  def body(i, carry):
    off = base + i * B
    pltpu.sync_copy(src_hbm.at[pl.ds(off, B)], sidx)
    pltpu.sync_copy(dst_hbm.at[pl.ds(off, B)], didx)
    pltpu.async_copy(h_hbm.at[sidx], rows, sem).wait()
    pltpu.sync_copy(rows, acc.at[didx], add=True)
    return carry

  lax.fori_loop(0, CHUNKS, body, 0)
  plsc.subcore_barrier()
  pltpu.sync_copy(acc.at[pl.ds(sid * RPT, RPT)],
                  out_hbm.at[pl.ds(cid * NP + sid * RPT, RPT)])


@functools.partial(
    pl.kernel,
    out_type=jax.ShapeDtypeStruct((NC * NP, H), jnp.float32),
    mesh=_MESH,
    scratch_types=[
        pltpu.VMEM((B,), jnp.int32),
        pltpu.VMEM((B,), jnp.int32),
        pltpu.VMEM((B, 2 * H), jnp.float32),
        pltpu.VMEM((B, H), jnp.float32),
        pltpu.VMEM((B, H), jnp.float32),
        pltpu.VMEM_SHARED((NP, H), jnp.float32),
        pltpu.SemaphoreType.DMA,
        pltpu.SemaphoreType.DMA,
    ],
)
def _sc_cagg(cu_hbm, f_hbm, src_hbm, dst_hbm, zeros_hbm, out_hbm,
             sidx, didx, cu_rows, f_rows, contrib, acc, sem1, sem2):
  cid = lax.axis_index("c")
  sid = lax.axis_index("s")
  wid = sid * NC + cid
  pltpu.sync_copy(zeros_hbm.at[pl.ds(sid * RPT, RPT)],
                  acc.at[pl.ds(sid * RPT, RPT)])
  plsc.subcore_barrier()
  base = wid * EPW

  def body(i, carry):
    off = base + i * B
    pltpu.sync_copy(src_hbm.at[pl.ds(off, B)], sidx)
    pltpu.sync_copy(dst_hbm.at[pl.ds(off, B)], didx)
    cp1 = pltpu.async_copy(cu_hbm.at[sidx], cu_rows, sem1)
    cp2 = pltpu.async_copy(f_hbm.at[didx], f_rows, sem2)
    cp1.wait()
    cp2.wait()

    def row(r, rc):
      for k in range(H // 16):
        sl = pl.ds(k * 16, 16)
        cs = cu_rows[r, sl]
        us = cu_rows[r, pl.ds(H + k * 16, 16)]
        fv = f_rows[r, sl]
        z = fv + us
        gate = 1.0 / (1.0 + jnp.exp(-z))
        contrib[r, sl] = gate * cs
      return rc

    lax.fori_loop(0, B, row, 0)
    pltpu.sync_copy(contrib, acc.at[didx], add=True)
    return carry

  lax.fori_loop(0, CHUNKS, body, 0)
  plsc.subcore_barrier()
  pltpu.sync_copy(acc.at[pl.ds(sid * RPT, RPT)],
                  out_hbm.at[pl.ds(cid * NP + sid * RPT, RPT)])


def _tc2_body(ioux_ref, hp_ref, cp_ref, uiou_ref, biou_ref,
              h_out_ref, c_out_ref):
  h_tild = hp_ref[0] + hp_ref[1]
  dn = (((1,), (1,)), ((), ()))
  sum_iou = lax.dot_general(h_tild, uiou_ref[...], dn,
                            preferred_element_type=jnp.float32)
  iou = ioux_ref[...] + sum_iou + biou_ref[...]
  i_g = jax.nn.sigmoid(iou[:, :H])
  o_g = jax.nn.sigmoid(iou[:, H:2 * H])
  u_g = jnp.tanh(iou[:, 2 * H:])
  c_agg = cp_ref[0] + cp_ref[1]
  c_out = i_g * u_g + c_agg
  c_out_ref[...] = c_out
  h_out_ref[...] = o_g * jnp.tanh(c_out)


def _tc2(iou_x, hp, cp, u_iou, b_iou):
  return pl.pallas_call(
      _tc2_body,
      grid=(GRID,),
      in_specs=[
          pl.BlockSpec((ROWS, 3 * H), lambda i: (i, 0)),
          pl.BlockSpec((NC, ROWS, H), lambda i: (0, i, 0)),
          pl.BlockSpec((NC, ROWS, H), lambda i: (0, i, 0)),
          pl.BlockSpec((3 * H, H), lambda i: (0, 0)),
          pl.BlockSpec((1, 3 * H), lambda i: (0, 0)),
      ],
      out_specs=[
          pl.BlockSpec((ROWS, H), lambda i: (i, 0)),
          pl.BlockSpec((ROWS, H), lambda i: (i, 0)),
      ],
      out_shape=[
          jax.ShapeDtypeStruct((N, H), jnp.float32),
          jax.ShapeDtypeStruct((N, H), jnp.float32),
      ],
  )(iou_x, hp, cp, u_iou, b_iou)


def kernel(x, h, c, edge_index, W_iouf, U_iou, U_f_w, U_f_b, b_iou):
  src = edge_index[0]
  dst = edge_index[1]
  iou_x, f_node, huf = _tc1(x, h, W_iouf, U_f_w, U_f_b.reshape(1, H))
  cu = jnp.concatenate([c, huf], axis=1)
  zeros = jnp.zeros((NP, H), jnp.float32)
  hp = _sc_htild(h, src, dst, zeros).reshape(NC, NP, H)
  cp = _sc_cagg(cu, f_node, src, dst, zeros).reshape(NC, NP, H)
  h_out, c_out = _tc2(iou_x, hp, cp, U_iou, b_iou.reshape(1, 3 * H))
  return (h_out, c_out)


# speedup vs baseline: 3.8912x; 2.6323x over previous
import functools

import jax
import jax.numpy as jnp
from jax import lax
from jax.experimental import pallas as pl
from jax.experimental.pallas import tpu as pltpu
from jax.experimental.pallas import tpu_sc as plsc

N = 10000
E = 320000
X = 128
H = 128

NC = 2
NS = 16
NW = NC * NS
EPW = E // NW

G = 32

BCA = 100
CHA = EPW // BCA
PRA = (CHA + G - 1) // G * G + G - CHA

BCB = 40
CHB = EPW // BCB
PRB = (CHB + G - 1) // G * G + G - CHB

NP = 10112
RPT = NP // NS

ROWS = 1000
GRID = N // ROWS


def _tc1_body(x_ref, h_ref, w_ref, uf_ref, ufb_ref, ioux_ref, nfnode_ref,
              huf_ref):
  dn = (((1,), (1,)), ((), ()))
  res = lax.dot_general(x_ref[...], w_ref[...], dn,
                        preferred_element_type=jnp.float32)
  ioux_ref[...] = res[:, : 3 * H]
  nfnode_ref[...] = -res[:, 3 * H:]
  huf_ref[...] = lax.dot_general(h_ref[...], uf_ref[...], dn,
                                 preferred_element_type=jnp.float32) \
      + ufb_ref[...]


def _tc1(x, h, w_iouf, u_f_w, u_f_b):
  return pl.pallas_call(
      _tc1_body,
      grid=(GRID,),
      in_specs=[
          pl.BlockSpec((ROWS, X), lambda i: (i, 0)),
          pl.BlockSpec((ROWS, H), lambda i: (i, 0)),
          pl.BlockSpec((4 * H, X), lambda i: (0, 0)),
          pl.BlockSpec((H, H), lambda i: (0, 0)),
          pl.BlockSpec((1, H), lambda i: (0, 0)),
      ],
      out_specs=[
          pl.BlockSpec((ROWS, 3 * H), lambda i: (i, 0)),
          pl.BlockSpec((ROWS, H), lambda i: (i, 0)),
          pl.BlockSpec((ROWS, H), lambda i: (i, 0)),
      ],
      out_shape=[
          jax.ShapeDtypeStruct((N, 3 * H), jnp.float32),
          jax.ShapeDtypeStruct((N, H), jnp.float32),
          jax.ShapeDtypeStruct((N, H), jnp.float32),
      ],
  )(x, h, w_iouf, u_f_w, u_f_b)


_MESH = plsc.VectorSubcoreMesh(core_axis_name="c", subcore_axis_name="s")


@functools.partial(
    pl.kernel,
    out_type=jax.ShapeDtypeStruct((NC * NP, H), jnp.float32),
    mesh=_MESH,
    scratch_types=[
        pltpu.VMEM((2, G, BCA), jnp.int32),
        pltpu.VMEM((2, G, BCA), jnp.int32),
        pltpu.VMEM((2, BCA, H), jnp.float32),
        pltpu.VMEM_SHARED((NP, H), jnp.float32),
        pltpu.SemaphoreType.DMA,
        pltpu.SemaphoreType.DMA,
    ],
)
def _sc_htild(h_hbm, src_hbm, dst_hbm, zeros_hbm, out_hbm,
              sidx, didx, rows, acc, sem0, sem1):
  cid = lax.axis_index("c")
  sid = lax.axis_index("s")
  wid = sid * NC + cid
  src_w = src_hbm.at[wid]
  dst_w = dst_hbm.at[wid]
  pltpu.sync_copy(src_w.at[pl.ds(0, G)], sidx.at[0])
  pltpu.sync_copy(dst_w.at[pl.ds(0, G)], didx.at[0])
  pltpu.sync_copy(zeros_hbm.at[pl.ds(sid * RPT, RPT)],
                  acc.at[pl.ds(sid * RPT, RPT)])
  plsc.subcore_barrier()

  sems = (sem0, sem1)
  for b in (0, 1):
    pltpu.async_copy(h_hbm.at[sidx.at[0, b]], rows.at[b], sems[b])

  def body(j, carry):
    for b in (0, 1):
      i = 2 * j + b
      nxt = i + 2
      gn = lax.div(nxt, G)
      pltpu.make_async_copy(h_hbm.at[sidx.at[0, 0]], rows.at[b],
                            sems[b]).wait()

      @pl.when(lax.rem(nxt, G) == 0)
      def _load_group():
        sl = lax.rem(gn, 2)
        pltpu.sync_copy(src_w.at[pl.ds(gn * G, G)], sidx.at[sl])
        pltpu.sync_copy(dst_w.at[pl.ds(gn * G, G)], didx.at[sl])

      pltpu.async_copy(h_hbm.at[sidx.at[lax.rem(gn, 2), lax.rem(nxt, G)]],
                       rows.at[b], sems[b])
      pltpu.sync_copy(rows.at[b],
                      acc.at[didx.at[lax.rem(lax.div(i, G), 2),
                                     lax.rem(i, G)]], add=True)
    return carry

  lax.fori_loop(0, CHA // 2, body, 0)
  for b in (0, 1):
    pltpu.make_async_copy(h_hbm.at[sidx.at[0, 0]], rows.at[b],
                          sems[b]).wait()
  plsc.subcore_barrier()
  pltpu.sync_copy(acc.at[pl.ds(sid * RPT, RPT)],
                  out_hbm.at[pl.ds(cid * NP + sid * RPT, RPT)])


@functools.partial(
    pl.kernel,
    out_type=jax.ShapeDtypeStruct((NC * NP, H), jnp.float32),
    mesh=_MESH,
    scratch_types=[
        pltpu.VMEM((2, G, BCB), jnp.int32),
        pltpu.VMEM((2, G, BCB), jnp.int32),
        pltpu.VMEM((2, BCB, 2 * H), jnp.float32),
        pltpu.VMEM((2, BCB, H), jnp.float32),
        pltpu.VMEM_SHARED((NP, H), jnp.float32),
        pltpu.SemaphoreType.DMA,
        pltpu.SemaphoreType.DMA,
        pltpu.SemaphoreType.DMA,
        pltpu.SemaphoreType.DMA,
    ],
)
def _sc_cagg(cu_hbm, nf_hbm, src_hbm, dst_hbm, zeros_hbm, out_hbm,
             sidx, didx, cu_rows, f_rows, acc,
             semc0, semc1, semf0, semf1):
  cid = lax.axis_index("c")
  sid = lax.axis_index("s")
  wid = sid * NC + cid
  src_w = src_hbm.at[wid]
  dst_w = dst_hbm.at[wid]
  pltpu.sync_copy(src_w.at[pl.ds(0, G)], sidx.at[0])
  pltpu.sync_copy(dst_w.at[pl.ds(0, G)], didx.at[0])
  pltpu.sync_copy(zeros_hbm.at[pl.ds(sid * RPT, RPT)],
                  acc.at[pl.ds(sid * RPT, RPT)])
  plsc.subcore_barrier()

  semc = (semc0, semc1)
  semf = (semf0, semf1)
  for b in (0, 1):
    pltpu.async_copy(cu_hbm.at[sidx.at[0, b]], cu_rows.at[b], semc[b])
    pltpu.async_copy(nf_hbm.at[didx.at[0, b]], f_rows.at[b], semf[b])

  def body(j, carry):
    for b in (0, 1):
      i = 2 * j + b
      nxt = i + 2
      gn = lax.div(nxt, G)
      pltpu.make_async_copy(cu_hbm.at[sidx.at[0, 0]], cu_rows.at[b],
                            semc[b]).wait()
      pltpu.make_async_copy(nf_hbm.at[didx.at[0, 0]], f_rows.at[b],
                            semf[b]).wait()
      cu_b = cu_rows.at[b]
      f_b = f_rows.at[b]

      @plsc.parallel_loop(0, BCB, 1, unroll=4)
      def row(r):
        for k in range(H // 16):
          sl = pl.ds(k * 16, 16)
          t = f_b[r, sl] - cu_b[r, pl.ds(H + k * 16, 16)]
          f_b[r, sl] = cu_b[r, sl] / (1.0 + jnp.exp(t))

      @pl.when(lax.rem(nxt, G) == 0)
      def _load_group():
        sl = lax.rem(gn, 2)
        pltpu.sync_copy(src_w.at[pl.ds(gn * G, G)], sidx.at[sl])
        pltpu.sync_copy(dst_w.at[pl.ds(gn * G, G)], didx.at[sl])

      pltpu.async_copy(cu_hbm.at[sidx.at[lax.rem(gn, 2), lax.rem(nxt, G)]],
                       cu_rows.at[b], semc[b])
      pltpu.async_copy(nf_hbm.at[didx.at[lax.rem(gn, 2), lax.rem(nxt, G)]],
                       f_rows.at[b], semf[b])
      pltpu.sync_copy(f_rows.at[b],
                      acc.at[didx.at[lax.rem(lax.div(i, G), 2),
                                     lax.rem(i, G)]], add=True)
    return carry

  lax.fori_loop(0, CHB // 2, body, 0)
  for b in (0, 1):
    pltpu.make_async_copy(cu_hbm.at[sidx.at[0, 0]], cu_rows.at[b],
                          semc[b]).wait()
    pltpu.make_async_copy(nf_hbm.at[didx.at[0, 0]], f_rows.at[b],
                          semf[b]).wait()
  plsc.subcore_barrier()
  pltpu.sync_copy(acc.at[pl.ds(sid * RPT, RPT)],
                  out_hbm.at[pl.ds(cid * NP + sid * RPT, RPT)])


def _tc2_body(ioux_ref, hp_ref, cp_ref, uiou_ref, biou_ref,
              h_out_ref, c_out_ref):
  h_tild = hp_ref[0] + hp_ref[1]
  dn = (((1,), (1,)), ((), ()))
  sum_iou = lax.dot_general(h_tild, uiou_ref[...], dn,
                            preferred_element_type=jnp.float32)
  iou = ioux_ref[...] + sum_iou + biou_ref[...]
  i_g = jax.nn.sigmoid(iou[:, :H])
  o_g = jax.nn.sigmoid(iou[:, H:2 * H])
  u_g = jnp.tanh(iou[:, 2 * H:])
  c_agg = cp_ref[0] + cp_ref[1]
  c_out = i_g * u_g + c_agg
  c_out_ref[...] = c_out
  h_out_ref[...] = o_g * jnp.tanh(c_out)


def _tc2(iou_x, hp, cp, u_iou, b_iou):
  return pl.pallas_call(
      _tc2_body,
      grid=(GRID,),
      in_specs=[
          pl.BlockSpec((ROWS, 3 * H), lambda i: (i, 0)),
          pl.BlockSpec((NC, ROWS, H), lambda i: (0, i, 0)),
          pl.BlockSpec((NC, ROWS, H), lambda i: (0, i, 0)),
          pl.BlockSpec((3 * H, H), lambda i: (0, 0)),
          pl.BlockSpec((1, 3 * H), lambda i: (0, 0)),
      ],
      out_specs=[
          pl.BlockSpec((ROWS, H), lambda i: (i, 0)),
          pl.BlockSpec((ROWS, H), lambda i: (i, 0)),
      ],
      out_shape=[
          jax.ShapeDtypeStruct((N, H), jnp.float32),
          jax.ShapeDtypeStruct((N, H), jnp.float32),
      ],
  )(iou_x, hp, cp, u_iou, b_iou)


def kernel(x, h, c, edge_index, W_iouf, U_iou, U_f_w, U_f_b, b_iou):
  src = edge_index[0]
  dst = edge_index[1]
  srcA = jnp.pad(src.reshape(NW, CHA, BCA), ((0, 0), (0, PRA), (0, 0)))
  dstA = jnp.pad(dst.reshape(NW, CHA, BCA), ((0, 0), (0, PRA), (0, 0)))
  srcB = jnp.pad(src.reshape(NW, CHB, BCB), ((0, 0), (0, PRB), (0, 0)))
  dstB = jnp.pad(dst.reshape(NW, CHB, BCB), ((0, 0), (0, PRB), (0, 0)))

  iou_x, nf_node, huf = _tc1(x, h, W_iouf, U_f_w, U_f_b.reshape(1, H))
  cu = jnp.concatenate([c, huf], axis=1)
  zeros = jnp.zeros((NP, H), jnp.float32)
  hp = _sc_htild(h, srcA, dstA, zeros).reshape(NC, NP, H)
  cp = _sc_cagg(cu, nf_node, srcB, dstB, zeros).reshape(NC, NP, H)
  h_out, c_out = _tc2(iou_x, hp, cp, U_iou, b_iou.reshape(1, 3 * H))
  return (h_out, c_out)
